# baseline (speedup 1.0000x reference)
import functools

import jax
import jax.numpy as jnp
from jax import lax
from jax.experimental import pallas as pl
from jax.experimental.pallas import tpu as pltpu
from jax.experimental.pallas import tpu_sc as plsc

N = 10000
E = 320000
H = 128

_NC = 2
_NS = 16
_NW = _NC * _NS
_SUB = 128
_CH_ROWS = 1
_CHUNK = _SUB * _CH_ROWS


def _ln(x, g, b):
    mu = jnp.mean(x, axis=-1, keepdims=True)
    var = jnp.mean((x - mu) ** 2, axis=-1, keepdims=True)
    return (x - mu) / jnp.sqrt(var + 1e-5) * g + b


def _dot(x, w):
    return jnp.dot(x, w, preferred_element_type=jnp.float32)


def _bdot(x, w):
    return jnp.dot(x.astype(jnp.bfloat16), w.astype(jnp.bfloat16),
                   preferred_element_type=jnp.float32)


def _full(shape):
    return pl.BlockSpec(shape, lambda i: tuple(0 for _ in shape))


def _mlp3_body(x_ref, w1, b1, w2, b2, w3, b3, *rest, ln):
    if ln:
        g, bb, o_ref = rest
    else:
        (o_ref,) = rest
    x = x_ref[...]
    h = jnp.maximum(_dot(x, w1[...]) + b1[...], 0.0)
    h = jnp.maximum(_dot(h, w2[...]) + b2[...], 0.0)
    y = _dot(h, w3[...]) + b3[...]
    if ln:
        y = _ln(y, g[...], bb[...])
    o_ref[...] = y


def _mlp3(x, p, block, ln):
    n, d = x.shape
    layers = p["layers"]
    args = [x]
    in_specs = [pl.BlockSpec((block, d), lambda i: (i, 0))]
    for lyr in layers:
        args += [lyr["W"], lyr["b"].reshape(1, -1)]
        in_specs += [_full(lyr["W"].shape), _full((1, lyr["b"].shape[0]))]
    if ln:
        args += [p["ln_g"].reshape(1, -1), p["ln_b"].reshape(1, -1)]
        in_specs += [_full((1, H)), _full((1, H))]
    out_dim = layers[-1]["W"].shape[1]
    return pl.pallas_call(
        functools.partial(_mlp3_body, ln=ln),
        grid=(n // block,),
        in_specs=in_specs,
        out_specs=pl.BlockSpec((block, out_dim), lambda i: (i, 0)),
        out_shape=jax.ShapeDtypeStruct((n, out_dim), jnp.float32),
    )(*args)


def _edge_body(e_ref, vs_ref, vd_ref, w1e, w1s, w1d, b1, w2, b2, w3, b3,
               g, bb, enew_ref, enext_ref):
    e = e_ref[...]
    h = (_bdot(e, w1e[...]) + _bdot(vs_ref[...], w1s[...])
         + _bdot(vd_ref[...], w1d[...]) + b1[...])
    h = jnp.maximum(h, 0.0)
    h = jnp.maximum(_bdot(h, w2[...]) + b2[...], 0.0)
    y = _bdot(h, w3[...]) + b3[...]
    y = _ln(y, g[...], bb[...])
    enew_ref[...] = y
    enext_ref[...] = e + y


def _edge_mlp(e, vg, p, block):
    layers = p["layers"]
    w1 = layers[0]["W"]
    nblk = E // block
    args = [e, vg, vg,
            w1[0:H], w1[H:2 * H], w1[2 * H:3 * H],
            layers[0]["b"].reshape(1, -1),
            layers[1]["W"], layers[1]["b"].reshape(1, -1),
            layers[2]["W"], layers[2]["b"].reshape(1, -1),
            p["ln_g"].reshape(1, -1), p["ln_b"].reshape(1, -1)]
    in_specs = [pl.BlockSpec((block, H), lambda i: (i, 0)),
                pl.BlockSpec((block, H), lambda i: (i, 0)),
                pl.BlockSpec((block, H), lambda i: (i + nblk, 0)),
                _full((H, H)), _full((H, H)), _full((H, H)), _full((1, H)),
                _full((H, H)), _full((1, H)),
                _full((H, H)), _full((1, H)),
                _full((1, H)), _full((1, H))]
    out = pl.pallas_call(
        _edge_body,
        grid=(nblk,),
        in_specs=in_specs,
        out_specs=[pl.BlockSpec((block, H), lambda i: (i, 0))] * 2,
        out_shape=[jax.ShapeDtypeStruct((E, H), jnp.float32)] * 2,
    )(*args)
    return out


def _node_body(v_ref, a0_ref, a1_ref, w1v, w1a, b1, w2, b2, w3, b3,
               g, bb, o_ref):
    v = v_ref[...]
    agg = a0_ref[0] + a1_ref[0]
    h = _dot(v, w1v[...]) + _dot(agg, w1a[...]) + b1[...]
    h = jnp.maximum(h, 0.0)
    h = jnp.maximum(_dot(h, w2[...]) + b2[...], 0.0)
    y = _dot(h, w3[...]) + b3[...]
    y = _ln(y, g[...], bb[...])
    o_ref[...] = v + y


def _node_mlp(v, agg, p, block):
    layers = p["layers"]
    w1 = layers[0]["W"]
    args = [v, agg, agg,
            w1[0:H], w1[H:2 * H],
            layers[0]["b"].reshape(1, -1),
            layers[1]["W"], layers[1]["b"].reshape(1, -1),
            layers[2]["W"], layers[2]["b"].reshape(1, -1),
            p["ln_g"].reshape(1, -1), p["ln_b"].reshape(1, -1)]
    in_specs = [pl.BlockSpec((block, H), lambda i: (i, 0)),
                pl.BlockSpec((1, block, H), lambda i: (0, i, 0)),
                pl.BlockSpec((1, block, H), lambda i: (1, i, 0)),
                _full((H, H)), _full((H, H)), _full((1, H)),
                _full((H, H)), _full((1, H)),
                _full((H, H)), _full((1, H)),
                _full((1, H)), _full((1, H))]
    return pl.pallas_call(
        _node_body,
        grid=(N // block,),
        in_specs=in_specs,
        out_specs=pl.BlockSpec((block, H), lambda i: (i, 0)),
        out_shape=jax.ShapeDtypeStruct((N, H), jnp.float32),
    )(*args)


@functools.lru_cache(maxsize=None)
def _sc_gather_kernel(n_sc, dtype_name):
    dtype = jnp.dtype(dtype_name)
    per_w = n_sc // _NW
    m = n_sc * _SUB
    mesh = plsc.VectorSubcoreMesh(core_axis_name="c", subcore_axis_name="s")

    @functools.partial(
        pl.kernel, mesh=mesh,
        out_type=jax.ShapeDtypeStruct((m, H), dtype),
        scratch_types=[pltpu.VMEM((2, 1, _SUB), jnp.int32),
                       pltpu.VMEM((2, _SUB, H), dtype),
                       pltpu.VMEM_SHARED((N, H), dtype),
                       pltpu.SemaphoreType.DMA,
                       pltpu.SemaphoreType.DMA],
    )
    def k(table_hbm, idx_hbm, out_hbm, idx_v, rows_v, tab, isem, osem):
        c = lax.axis_index("c")
        s = lax.axis_index("s")
        wid = s * _NC + c
        base = wid * per_w
        nq = N // 80
        for q_i in range(-(-nq // _NS)):
            q = s + q_i * _NS

            @pl.when(q < nq)
            def _(q=q):
                pltpu.sync_copy(table_hbm.at[pl.ds(q * 80, 80)],
                                tab.at[pl.ds(q * 80, 80)])
        plsc.subcore_barrier()
        pltpu.async_copy(idx_hbm.at[base], idx_v.at[0], isem)

        def body(it, carry):
            p = lax.rem(it, 2)
            sc = base + it
            pltpu.make_async_copy(idx_hbm.at[sc], idx_v.at[p], isem).wait()

            @pl.when(it + 1 < per_w)
            def _():
                pltpu.async_copy(idx_hbm.at[sc + 1], idx_v.at[1 - p], isem)

            @pl.when(it >= 2)
            def _():
                pltpu.make_async_copy(
                    rows_v.at[p],
                    out_hbm.at[pl.ds((sc - 2) * _SUB, _SUB)],
                    osem).wait()

            pltpu.sync_copy(tab.at[idx_v.at[p, 0]], rows_v.at[p])
            pltpu.async_copy(rows_v.at[p],
                             out_hbm.at[pl.ds(sc * _SUB, _SUB)], osem)
            return carry

        lax.fori_loop(0, per_w, body, 0)
        for tail in (per_w - 2, per_w - 1):
            pltpu.make_async_copy(
                rows_v.at[tail % 2],
                out_hbm.at[pl.ds((base + tail) * _SUB, _SUB)],
                osem).wait()

    return k


def _sc_gather(table, idx3):
    return _sc_gather_kernel(idx3.shape[0], table.dtype.name)(table, idx3)


def _sc_segsum(rows, dst3):
    return _sc_segsum_kernel(dst3.shape[0])(rows, dst3)


@functools.lru_cache(maxsize=None)
def _sc_segsum_kernel(n_sc):
    per_w = -(-n_sc // _NW)
    mesh = plsc.VectorSubcoreMesh(core_axis_name="c", subcore_axis_name="s")

    @functools.partial(
        pl.kernel, mesh=mesh,
        out_type=jax.ShapeDtypeStruct((_NC, N, H), jnp.float32),
        scratch_types=[pltpu.VMEM((_CH_ROWS, _SUB), jnp.int32),
                       pltpu.VMEM((_CHUNK, H), jnp.float32),
                       pltpu.VMEM_SHARED((N, H), jnp.float32)],
    )
    def k(rows_hbm, dst_hbm, out_hbm, idx_v, rows_v, acc):
        c = lax.axis_index("c")
        s = lax.axis_index("s")
        wid = s * _NC + c

        def zbody(r, carry):
            for vv in range(H // 16):
                rows_v[r, pl.ds(vv * 16, 16)] = jnp.zeros(
                    (16,), jnp.float32)
            return carry

        lax.fori_loop(0, 80, zbody, 0)
        nz = N // 80
        for z_i in range(-(-nz // _NS)):
            z = s + z_i * _NS

            @pl.when(z < nz)
            def _(z=z):
                pltpu.sync_copy(rows_v.at[pl.ds(0, 80)],
                                acc.at[pl.ds(z * 80, 80)])
        plsc.subcore_barrier()

        def body(it, carry):
            sc = wid + it * _NW

            @pl.when(sc < n_sc)
            def _():
                pltpu.sync_copy(dst_hbm.at[sc], idx_v)
                pltpu.sync_copy(rows_hbm.at[pl.ds(sc * _CHUNK, _CHUNK)],
                                rows_v)
                for j in range(_CH_ROWS):
                    pltpu.sync_copy(rows_v.at[pl.ds(j * _SUB, _SUB)],
                                    acc.at[idx_v.at[j]], add=True)

            return carry

        lax.fori_loop(0, per_w, body, 0)
        plsc.subcore_barrier()
        for q_i in range(-(-nz // _NS)):
            q = s + q_i * _NS

            @pl.when(q < nz)
            def _(q=q):
                pltpu.sync_copy(acc.at[pl.ds(q * 80, 80)],
                                out_hbm.at[c, pl.ds(q * 80, 80)])

    return k


def kernel(nodes, edge_attr, edge_index, params):
    src = edge_index[0].astype(jnp.int32)
    dst = edge_index[1].astype(jnp.int32)
    n_sc_g = -(-2 * E // (_SUB * _NW)) * _NW
    pad = n_sc_g * _SUB - 2 * E
    idx_all = jnp.concatenate(
        [src, dst, jnp.zeros((pad,), jnp.int32)]).reshape(-1, 1, _SUB)
    dst3 = dst.reshape(-1, _CH_ROWS, _SUB)

    v = _mlp3(nodes, params["enc_node"], block=1000, ln=True)
    e = _mlp3(edge_attr, params["enc_edge"], block=1280, ln=True)
    for step in params["proc"]:
        vg = _sc_gather(v, idx_all)
        e_new, e = _edge_mlp(e, vg, step["phi_edge"], block=1280)
        agg = _sc_segsum(e_new, dst3)
        v = _node_mlp(v, agg, step["phi_node"], block=1000)
    return _mlp3(v, params["dec"], block=1000, ln=False)


# speedup vs baseline: 1.0280x; 1.0280x over previous
import functools

import jax
import jax.numpy as jnp
from jax import lax
from jax.experimental import pallas as pl
from jax.experimental.pallas import tpu as pltpu
from jax.experimental.pallas import tpu_sc as plsc

N = 10000
E = 320000
H = 128

_NC = 2
_NS = 16
_NW = _NC * _NS
_SUB = 128
_CH_ROWS = 1
_CHUNK = _SUB * _CH_ROWS


def _ln(x, g, b):
    mu = jnp.mean(x, axis=-1, keepdims=True)
    var = jnp.mean((x - mu) ** 2, axis=-1, keepdims=True)
    return (x - mu) / jnp.sqrt(var + 1e-5) * g + b


def _dot(x, w):
    return jnp.dot(x, w, preferred_element_type=jnp.float32)


def _bdot(x, w):
    return jnp.dot(x.astype(jnp.bfloat16), w.astype(jnp.bfloat16),
                   preferred_element_type=jnp.float32)


def _full(shape):
    return pl.BlockSpec(shape, lambda i: tuple(0 for _ in shape))


def _mlp3_body(x_ref, w1, b1, w2, b2, w3, b3, *rest, ln, out_dtype):
    if ln:
        g, bb, o_ref = rest
    else:
        (o_ref,) = rest
    x = x_ref[...]
    h = jnp.maximum(_dot(x, w1[...]) + b1[...], 0.0)
    h = jnp.maximum(_dot(h, w2[...]) + b2[...], 0.0)
    y = _dot(h, w3[...]) + b3[...]
    if ln:
        y = _ln(y, g[...], bb[...])
    o_ref[...] = y.astype(out_dtype)


def _mlp3(x, p, block, ln, out_dtype=jnp.float32):
    n, d = x.shape
    layers = p["layers"]
    args = [x]
    in_specs = [pl.BlockSpec((block, d), lambda i: (i, 0))]
    for lyr in layers:
        args += [lyr["W"], lyr["b"].reshape(1, -1)]
        in_specs += [_full(lyr["W"].shape), _full((1, lyr["b"].shape[0]))]
    if ln:
        args += [p["ln_g"].reshape(1, -1), p["ln_b"].reshape(1, -1)]
        in_specs += [_full((1, H)), _full((1, H))]
    out_dim = layers[-1]["W"].shape[1]
    return pl.pallas_call(
        functools.partial(_mlp3_body, ln=ln, out_dtype=out_dtype),
        grid=(n // block,),
        in_specs=in_specs,
        out_specs=pl.BlockSpec((block, out_dim), lambda i: (i, 0)),
        out_shape=jax.ShapeDtypeStruct((n, out_dim), out_dtype),
    )(*args)


def _edge_body(e_ref, vs_ref, vd_ref, w1e, w1s, w1d, b1, w2, b2, w3, b3,
               g, bb, enew_ref, enext_ref):
    e = e_ref[...]
    h = (_bdot(e, w1e[...]) + _bdot(vs_ref[...], w1s[...])
         + _bdot(vd_ref[...], w1d[...]) + b1[...])
    h = jnp.maximum(h, 0.0)
    h = jnp.maximum(_bdot(h, w2[...]) + b2[...], 0.0)
    y = _bdot(h, w3[...]) + b3[...]
    y = _ln(y, g[...], bb[...])
    enew_ref[...] = y
    enext_ref[...] = (e.astype(jnp.float32) + y).astype(jnp.bfloat16)


def _edge_mlp(e, vg, p, block):
    layers = p["layers"]
    w1 = layers[0]["W"]
    nblk = E // block
    args = [e, vg, vg,
            w1[0:H], w1[H:2 * H], w1[2 * H:3 * H],
            layers[0]["b"].reshape(1, -1),
            layers[1]["W"], layers[1]["b"].reshape(1, -1),
            layers[2]["W"], layers[2]["b"].reshape(1, -1),
            p["ln_g"].reshape(1, -1), p["ln_b"].reshape(1, -1)]
    in_specs = [pl.BlockSpec((block, H), lambda i: (i, 0)),
                pl.BlockSpec((block, H), lambda i: (i, 0)),
                pl.BlockSpec((block, H), lambda i: (i + nblk, 0)),
                _full((H, H)), _full((H, H)), _full((H, H)), _full((1, H)),
                _full((H, H)), _full((1, H)),
                _full((H, H)), _full((1, H)),
                _full((1, H)), _full((1, H))]
    out = pl.pallas_call(
        _edge_body,
        grid=(nblk,),
        in_specs=in_specs,
        out_specs=[pl.BlockSpec((block, H), lambda i: (i, 0))] * 2,
        out_shape=[jax.ShapeDtypeStruct((E, H), jnp.float32),
                   jax.ShapeDtypeStruct((E, H), jnp.bfloat16)],
    )(*args)
    return out


def _node_body(v_ref, a0_ref, a1_ref, w1v, w1a, b1, w2, b2, w3, b3,
               g, bb, o_ref):
    v = v_ref[...]
    agg = a0_ref[0] + a1_ref[0]
    h = _dot(v, w1v[...]) + _dot(agg, w1a[...]) + b1[...]
    h = jnp.maximum(h, 0.0)
    h = jnp.maximum(_dot(h, w2[...]) + b2[...], 0.0)
    y = _dot(h, w3[...]) + b3[...]
    y = _ln(y, g[...], bb[...])
    o_ref[...] = v + y


def _node_mlp(v, agg, p, block):
    layers = p["layers"]
    w1 = layers[0]["W"]
    args = [v, agg, agg,
            w1[0:H], w1[H:2 * H],
            layers[0]["b"].reshape(1, -1),
            layers[1]["W"], layers[1]["b"].reshape(1, -1),
            layers[2]["W"], layers[2]["b"].reshape(1, -1),
            p["ln_g"].reshape(1, -1), p["ln_b"].reshape(1, -1)]
    in_specs = [pl.BlockSpec((block, H), lambda i: (i, 0)),
                pl.BlockSpec((1, block, H), lambda i: (0, i, 0)),
                pl.BlockSpec((1, block, H), lambda i: (1, i, 0)),
                _full((H, H)), _full((H, H)), _full((1, H)),
                _full((H, H)), _full((1, H)),
                _full((H, H)), _full((1, H)),
                _full((1, H)), _full((1, H))]
    return pl.pallas_call(
        _node_body,
        grid=(N // block,),
        in_specs=in_specs,
        out_specs=pl.BlockSpec((block, H), lambda i: (i, 0)),
        out_shape=jax.ShapeDtypeStruct((N, H), jnp.float32),
    )(*args)


@functools.lru_cache(maxsize=None)
def _sc_gather_kernel(n_sc, dtype_name):
    dtype = jnp.dtype(dtype_name)
    per_w = n_sc // _NW
    m = n_sc * _SUB
    mesh = plsc.VectorSubcoreMesh(core_axis_name="c", subcore_axis_name="s")

    @functools.partial(
        pl.kernel, mesh=mesh,
        out_type=jax.ShapeDtypeStruct((m, H), dtype),
        scratch_types=[pltpu.VMEM((2, 1, _SUB), jnp.int32),
                       pltpu.VMEM((2, _SUB, H), dtype),
                       pltpu.VMEM_SHARED((N, H), dtype),
                       pltpu.SemaphoreType.DMA,
                       pltpu.SemaphoreType.DMA],
    )
    def k(table_hbm, idx_hbm, out_hbm, idx_v, rows_v, tab, isem, osem):
        c = lax.axis_index("c")
        s = lax.axis_index("s")
        wid = s * _NC + c
        base = wid * per_w
        nq = N // 80
        for q_i in range(-(-nq // _NS)):
            q = s + q_i * _NS

            @pl.when(q < nq)
            def _(q=q):
                pltpu.sync_copy(table_hbm.at[pl.ds(q * 80, 80)],
                                tab.at[pl.ds(q * 80, 80)])
        plsc.subcore_barrier()
        pltpu.async_copy(idx_hbm.at[base], idx_v.at[0], isem)

        def body(it, carry):
            p = lax.rem(it, 2)
            sc = base + it
            pltpu.make_async_copy(idx_hbm.at[sc], idx_v.at[p], isem).wait()

            @pl.when(it + 1 < per_w)
            def _():
                pltpu.async_copy(idx_hbm.at[sc + 1], idx_v.at[1 - p], isem)

            @pl.when(it >= 2)
            def _():
                pltpu.make_async_copy(
                    rows_v.at[p],
                    out_hbm.at[pl.ds((sc - 2) * _SUB, _SUB)],
                    osem).wait()

            pltpu.sync_copy(tab.at[idx_v.at[p, 0]], rows_v.at[p])
            pltpu.async_copy(rows_v.at[p],
                             out_hbm.at[pl.ds(sc * _SUB, _SUB)], osem)
            return carry

        lax.fori_loop(0, per_w, body, 0)
        for tail in (per_w - 2, per_w - 1):
            pltpu.make_async_copy(
                rows_v.at[tail % 2],
                out_hbm.at[pl.ds((base + tail) * _SUB, _SUB)],
                osem).wait()

    return k


def _sc_gather(table, idx3):
    return _sc_gather_kernel(idx3.shape[0], table.dtype.name)(table, idx3)


def _sc_segsum(rows, dst3):
    return _sc_segsum_kernel(dst3.shape[0])(rows, dst3)


@functools.lru_cache(maxsize=None)
def _sc_segsum_kernel(n_sc):
    per_w = -(-n_sc // _NW)
    mesh = plsc.VectorSubcoreMesh(core_axis_name="c", subcore_axis_name="s")

    @functools.partial(
        pl.kernel, mesh=mesh,
        out_type=jax.ShapeDtypeStruct((_NC, N, H), jnp.float32),
        scratch_types=[pltpu.VMEM((_CH_ROWS, _SUB), jnp.int32),
                       pltpu.VMEM((_CHUNK, H), jnp.float32),
                       pltpu.VMEM_SHARED((N, H), jnp.float32)],
    )
    def k(rows_hbm, dst_hbm, out_hbm, idx_v, rows_v, acc):
        c = lax.axis_index("c")
        s = lax.axis_index("s")
        wid = s * _NC + c

        def zbody(r, carry):
            for vv in range(H // 16):
                rows_v[r, pl.ds(vv * 16, 16)] = jnp.zeros(
                    (16,), jnp.float32)
            return carry

        lax.fori_loop(0, 80, zbody, 0)
        nz = N // 80
        for z_i in range(-(-nz // _NS)):
            z = s + z_i * _NS

            @pl.when(z < nz)
            def _(z=z):
                pltpu.sync_copy(rows_v.at[pl.ds(0, 80)],
                                acc.at[pl.ds(z * 80, 80)])
        plsc.subcore_barrier()

        def body(it, carry):
            sc = wid + it * _NW

            @pl.when(sc < n_sc)
            def _():
                pltpu.sync_copy(dst_hbm.at[sc], idx_v)
                pltpu.sync_copy(rows_hbm.at[pl.ds(sc * _CHUNK, _CHUNK)],
                                rows_v)
                for j in range(_CH_ROWS):
                    pltpu.sync_copy(rows_v.at[pl.ds(j * _SUB, _SUB)],
                                    acc.at[idx_v.at[j]], add=True)

            return carry

        lax.fori_loop(0, per_w, body, 0)
        plsc.subcore_barrier()
        for q_i in range(-(-nz // _NS)):
            q = s + q_i * _NS

            @pl.when(q < nz)
            def _(q=q):
                pltpu.sync_copy(acc.at[pl.ds(q * 80, 80)],
                                out_hbm.at[c, pl.ds(q * 80, 80)])

    return k


def kernel(nodes, edge_attr, edge_index, params):
    src = edge_index[0].astype(jnp.int32)
    dst = edge_index[1].astype(jnp.int32)
    n_sc_g = -(-2 * E // (_SUB * _NW)) * _NW
    pad = n_sc_g * _SUB - 2 * E
    idx_all = jnp.concatenate(
        [src, dst, jnp.zeros((pad,), jnp.int32)]).reshape(-1, 1, _SUB)
    dst3 = dst.reshape(-1, _CH_ROWS, _SUB)

    v = _mlp3(nodes, params["enc_node"], block=1000, ln=True)
    e = _mlp3(edge_attr, params["enc_edge"], block=1280, ln=True)
    for step in params["proc"]:
        vg = _sc_gather(v, idx_all)
        e_new, e = _edge_mlp(e, vg, step["phi_edge"], block=1280)
        agg = _sc_segsum(e_new, dst3)
        v = _node_mlp(v, agg, step["phi_node"], block=1000)
    return _mlp3(v, params["dec"], block=1000, ln=False)
